# baseline (speedup 1.0000x reference)
import functools

import jax
import jax.numpy as jnp
from jax import lax
from jax.experimental import pallas as pl
from jax.experimental.pallas import tpu as pltpu

P = 32768
NL = 3
N = P * NL
NG = 256
DYN = 4
STAT = 5
NHL = 6
D = 128
E = 1048576
F = 16
BLK = 2048

def _stat_mask():
    i = lax.broadcasted_iota(jnp.int32, (1, F), 1)
    return jnp.where((i >= DYN) & (i < F - 1), 1.0, 0.0).astype(jnp.float32)

MODE_NONE = 0
MODE_SUB = 1
MODE_MASK = 2


def _mlp_gin_body(eps_ref, h_ref, pp_ref, w1, b1, w2, b2, w3, b3, w4, b4,
                  out_ref):
    u = (1.0 + eps_ref[0]) * h_ref[...] + pp_ref[0] + pp_ref[1]
    a = jnp.maximum(jnp.dot(u, w1[...], preferred_element_type=jnp.float32, precision=lax.Precision.HIGHEST) + b1[...], 0.0)
    a = jnp.maximum(jnp.dot(a, w2[...], preferred_element_type=jnp.float32, precision=lax.Precision.HIGHEST) + b2[...], 0.0)
    a = jnp.maximum(jnp.dot(a, w3[...], preferred_element_type=jnp.float32, precision=lax.Precision.HIGHEST) + b3[...], 0.0)
    o = jnp.maximum(jnp.dot(a, w4[...], preferred_element_type=jnp.float32, precision=lax.Precision.HIGHEST) + b4[...], 0.0)
    out_ref[...] = o + h_ref[...] * _stat_mask()


def _mlp_plain_body(h_ref, w1, b1, w2, b2, w3, b3, w4, b4, out_ref):
    a = jnp.maximum(jnp.dot(h_ref[...], w1[...], preferred_element_type=jnp.float32, precision=lax.Precision.HIGHEST) + b1[...], 0.0)
    a = jnp.maximum(jnp.dot(a, w2[...], preferred_element_type=jnp.float32, precision=lax.Precision.HIGHEST) + b2[...], 0.0)
    a = jnp.maximum(jnp.dot(a, w3[...], preferred_element_type=jnp.float32, precision=lax.Precision.HIGHEST) + b3[...], 0.0)
    o = jnp.maximum(jnp.dot(a, w4[...], preferred_element_type=jnp.float32, precision=lax.Precision.HIGHEST) + b4[...], 0.0)
    out_ref[...] = o + h_ref[...] * _stat_mask()


def _w_specs(ws):
    return [pl.BlockSpec(w.shape, lambda i, n=w.ndim: (0,) * n) for w in ws]


def _run_mlp_gin(H, pp, eps, ws, base):
    b0 = base // BLK
    row_spec = pl.BlockSpec((BLK, F), lambda i, b0=b0: (b0 + i, 0))
    return pl.pallas_call(
        _mlp_gin_body,
        grid=(P // BLK,),
        in_specs=[pl.BlockSpec(memory_space=pltpu.SMEM),
                  row_spec,
                  pl.BlockSpec((2, BLK, F), lambda i: (0, i, 0))]
                 + _w_specs(ws),
        out_specs=row_spec,
        out_shape=jax.ShapeDtypeStruct((N, F), jnp.float32),
        input_output_aliases={1: 0},
    )(eps, H, pp, *ws)


def _run_mlp_plain(H, ws, base):
    b0 = base // BLK
    row_spec = pl.BlockSpec((BLK, F), lambda i, b0=b0: (b0 + i, 0))
    return pl.pallas_call(
        _mlp_plain_body,
        grid=(P // BLK,),
        in_specs=[row_spec] + _w_specs(ws),
        out_specs=row_spec,
        out_shape=jax.ShapeDtypeStruct((N, F), jnp.float32),
        input_output_aliases={0: 0},
    )(H, *ws)


def _init_h_body(x_ref, fs_ref, hlp_ref, out_ref):
    col = lax.broadcasted_iota(jnp.int32, (1, F), 1)
    e0 = jnp.where(col == 0, 1.0, 0.0).astype(jnp.float32)
    r = lax.broadcasted_iota(jnp.int32, (STAT, F), 0)
    c = lax.broadcasted_iota(jnp.int32, (STAT, F), 1)
    s = jnp.where(c == r + DYN, 1.0, 0.0).astype(jnp.float32)
    out_ref[...] = (x_ref[...] * e0
                    + jnp.dot(fs_ref[...], s, preferred_element_type=jnp.float32, precision=lax.Precision.HIGHEST)
                    + hlp_ref[...])


def _run_init_h(x, fstat, hlp):
    return pl.pallas_call(
        _init_h_body,
        grid=(N // BLK,),
        in_specs=[pl.BlockSpec((BLK, 1), lambda i: (i, 0)),
                  pl.BlockSpec((BLK, STAT), lambda i: (i, 0)),
                  pl.BlockSpec((BLK, F), lambda i: (i, 0))],
        out_specs=pl.BlockSpec((BLK, F), lambda i: (i, 0)),
        out_shape=jax.ShapeDtypeStruct((N, F), jnp.float32),
    )(x, fstat, hlp)


def _final_body(hl_ref, pp_ref, w1a, w1b, b1, w2, b2, w3, b3, w4, b4, out_ref):
    pooled = pp_ref[0] + pp_ref[1]
    a = jnp.maximum(jnp.dot(hl_ref[...], w1a[...], preferred_element_type=jnp.float32, precision=lax.Precision.HIGHEST)
                    + jnp.dot(pooled, w1b[...], preferred_element_type=jnp.float32, precision=lax.Precision.HIGHEST)
                    + b1[...], 0.0)
    a = jnp.maximum(jnp.dot(a, w2[...], preferred_element_type=jnp.float32, precision=lax.Precision.HIGHEST) + b2[...], 0.0)
    a = jnp.maximum(jnp.dot(a, w3[...], preferred_element_type=jnp.float32, precision=lax.Precision.HIGHEST) + b3[...], 0.0)
    out_ref[...] = jnp.maximum(jnp.dot(a, w4[...], preferred_element_type=jnp.float32, precision=lax.Precision.HIGHEST) + b4[...], 0.0)


def _run_final(hl16, pool_pp, ws):
    ins = [hl16, pool_pp] + list(ws)
    return pl.pallas_call(
        _final_body,
        in_specs=[pl.BlockSpec(v.shape, lambda *_, n=v.ndim: (0,) * n) for v in ins],
        out_specs=pl.BlockSpec((NG, D), lambda *_: (0, 0)),
        out_shape=jax.ShapeDtypeStruct((NG, D), jnp.float32),
    )(*ins)


def _agg(H, src_g, dst, mode):
    g = H[src_g]
    if mode == MODE_SUB:
        d = dst - P
        d = jnp.where(d >= 0, d, P)
    elif mode == MODE_MASK:
        d = jnp.where(dst < P, dst, P)
    else:
        d = dst
    part = jnp.zeros((P + 1, F), jnp.float32).at[d].add(g)[:P]
    return jnp.stack([part, jnp.zeros_like(part)])


def _pool(H, batch_idx):
    part = jax.ops.segment_sum(H, batch_idx, num_segments=NG)
    return jnp.stack([part, jnp.zeros_like(part)])


def _pad_mlp(layers):
    (w1, b1), (w2, b2), (w3, b3), (w4, b4) = layers
    w1p = jnp.zeros((F, D), jnp.float32).at[:w1.shape[0]].set(w1)
    w4p = jnp.zeros((D, F), jnp.float32).at[:, :DYN].set(w4)
    b4p = jnp.zeros((F,), jnp.float32).at[:DYN].set(b4)
    return (w1p, b1[None, :], w2, b2[None, :], w3, b3[None, :], w4p, b4p[None, :])


def kernel(x, feature_mtx_static, hlvs, params, batch_idx, inner_edges,
           forward_edges, backward_edges):
    ws_in = _pad_mlp(params['inlayer_mlp'])
    ws_fw = _pad_mlp(params['forward_mlp'])
    ws_bw = _pad_mlp(params['backward_mlp'])
    ws_nd = _pad_mlp(params['node_dnn'])
    eps_in = (params['inlayer_eps'])[None].astype(jnp.float32)
    eps_fw = (params['forward_eps'])[None].astype(jnp.float32)
    eps_bw = (params['backward_eps'])[None].astype(jnp.float32)

    tbl = jnp.zeros((NG, F), jnp.float32).at[:, DYN + STAT:DYN + STAT + NHL].set(hlvs)
    hlp = tbl[batch_idx]
    H = _run_init_h(x, feature_mtx_static, hlp)

    def conv(H, edges, lo, mode, ws, eps, base):
        src_g = edges[0] + lo
        pp = _agg(H, src_g, edges[1], mode)
        return _run_mlp_gin(H, pp, eps, ws, base)

    H = conv(H, inner_edges[0], 0, MODE_NONE, ws_in, eps_in, 0)
    H = conv(H, forward_edges[0], 0, MODE_SUB, ws_fw, eps_fw, P)
    H = _run_mlp_plain(H, ws_nd, P)
    H = conv(H, inner_edges[1], P, MODE_NONE, ws_in, eps_in, P)
    H = conv(H, forward_edges[1], P, MODE_SUB, ws_fw, eps_fw, 2 * P)
    H = _run_mlp_plain(H, ws_nd, 2 * P)
    H = conv(H, inner_edges[2], 2 * P, MODE_NONE, ws_in, eps_in, 2 * P)
    H = conv(H, backward_edges[1], P, MODE_MASK, ws_bw, eps_bw, P)
    H = conv(H, inner_edges[1], P, MODE_NONE, ws_in, eps_in, P)
    H = _run_mlp_plain(H, ws_nd, P)
    H = conv(H, backward_edges[0], 0, MODE_MASK, ws_bw, eps_bw, 0)
    H = conv(H, inner_edges[0], 0, MODE_NONE, ws_in, eps_in, 0)
    H = _run_mlp_plain(H, ws_nd, 0)

    pool_pp = _pool(H, batch_idx)

    (w1, b1), (w2, b2), (w3, b3), (w4, b4) = params['hlv_dnn']
    w1a = jnp.zeros((F, D), jnp.float32).at[:NHL].set(w1[:NHL])
    w1b = jnp.zeros((F, D), jnp.float32).at[:DYN].set(w1[NHL:NHL + DYN])
    w4p = jnp.zeros((D, D), jnp.float32).at[:, :1].set(w4)
    hl16 = jnp.zeros((NG, F), jnp.float32).at[:, :NHL].set(hlvs)
    fin = _run_final(hl16, pool_pp,
                     (w1a, w1b, b1[None, :], w2, b2[None, :], w3, b3[None, :],
                      w4p, jnp.zeros((1, D), jnp.float32).at[0, 0].set(b4[0])))
    return fin[:, :1]


# speedup vs baseline: 15.2551x; 15.2551x over previous
import functools

import jax
import jax.numpy as jnp
from jax import lax
from jax.experimental import pallas as pl
from jax.experimental.pallas import tpu as pltpu
from jax.experimental.pallas import tpu_sc as plsc

P = 32768
NL = 3
N = P * NL
NG = 256
DYN = 4
STAT = 5
NHL = 6
D = 128
E = 1048576
F = 16
BLK = 2048

def _bdot(a, b):
    return jnp.dot(a.astype(jnp.bfloat16), b.astype(jnp.bfloat16),
                   preferred_element_type=jnp.float32)


def _stat_mask():
    i = lax.broadcasted_iota(jnp.int32, (1, F), 1)
    return jnp.where((i >= DYN) & (i < F - 1), 1.0, 0.0).astype(jnp.float32)

MODE_NONE = 0
MODE_SUB = 1
MODE_MASK = 2


def _mlp_gin_body(eps_ref, h_ref, pp_ref, w1, b1, w2, b2, w3, b3, w4, b4,
                  out_ref):
    u = (1.0 + eps_ref[0]) * h_ref[...] + pp_ref[0] + pp_ref[1]
    a = jnp.maximum(_bdot(u, w1[...]) + b1[...], 0.0)
    a = jnp.maximum(_bdot(a, w2[...]) + b2[...], 0.0)
    a = jnp.maximum(_bdot(a, w3[...]) + b3[...], 0.0)
    o = jnp.maximum(_bdot(a, w4[...]) + b4[...], 0.0)
    out_ref[...] = o + h_ref[...] * _stat_mask()


def _mlp_plain_body(h_ref, w1, b1, w2, b2, w3, b3, w4, b4, out_ref):
    a = jnp.maximum(_bdot(h_ref[...], w1[...]) + b1[...], 0.0)
    a = jnp.maximum(_bdot(a, w2[...]) + b2[...], 0.0)
    a = jnp.maximum(_bdot(a, w3[...]) + b3[...], 0.0)
    o = jnp.maximum(_bdot(a, w4[...]) + b4[...], 0.0)
    out_ref[...] = o + h_ref[...] * _stat_mask()


def _w_specs(ws):
    return [pl.BlockSpec(w.shape, lambda i, n=w.ndim: (0,) * n) for w in ws]


def _run_mlp_gin(H, pp, eps, ws, base):
    b0 = base // BLK
    row_spec = pl.BlockSpec((BLK, F), lambda i, b0=b0: (b0 + i, 0))
    return pl.pallas_call(
        _mlp_gin_body,
        grid=(P // BLK,),
        in_specs=[pl.BlockSpec(memory_space=pltpu.SMEM),
                  row_spec,
                  pl.BlockSpec((2, BLK, F), lambda i: (0, i, 0))]
                 + _w_specs(ws),
        out_specs=row_spec,
        out_shape=jax.ShapeDtypeStruct((N, F), jnp.float32),
        input_output_aliases={1: 0},
    )(eps, H, pp, *ws)


def _run_mlp_plain(H, ws, base):
    b0 = base // BLK
    row_spec = pl.BlockSpec((BLK, F), lambda i, b0=b0: (b0 + i, 0))
    return pl.pallas_call(
        _mlp_plain_body,
        grid=(P // BLK,),
        in_specs=[row_spec] + _w_specs(ws),
        out_specs=row_spec,
        out_shape=jax.ShapeDtypeStruct((N, F), jnp.float32),
        input_output_aliases={0: 0},
    )(H, *ws)


def _init_h_body(x_ref, fs_ref, hlp_ref, out_ref):
    col = lax.broadcasted_iota(jnp.int32, (1, F), 1)
    e0 = jnp.where(col == 0, 1.0, 0.0).astype(jnp.float32)
    r = lax.broadcasted_iota(jnp.int32, (STAT, F), 0)
    c = lax.broadcasted_iota(jnp.int32, (STAT, F), 1)
    s = jnp.where(c == r + DYN, 1.0, 0.0).astype(jnp.float32)
    out_ref[...] = (x_ref[...] * e0
                    + jnp.dot(fs_ref[...], s, preferred_element_type=jnp.float32, precision=lax.Precision.HIGHEST)
                    + hlp_ref[...])


def _run_init_h(x, fstat, hlp):
    return pl.pallas_call(
        _init_h_body,
        grid=(N // BLK,),
        in_specs=[pl.BlockSpec((BLK, 1), lambda i: (i, 0)),
                  pl.BlockSpec((BLK, STAT), lambda i: (i, 0)),
                  pl.BlockSpec((BLK, F), lambda i: (i, 0))],
        out_specs=pl.BlockSpec((BLK, F), lambda i: (i, 0)),
        out_shape=jax.ShapeDtypeStruct((N, F), jnp.float32),
    )(x, fstat, hlp)


def _final_body(hl_ref, pp_ref, shift, w1, b1, w2, b2, w3, b3, w4, b4, out_ref):
    pooled = pp_ref[0] + pp_ref[1]
    u = hl_ref[...] + jnp.dot(pooled, shift[...],
                              preferred_element_type=jnp.float32,
                              precision=lax.Precision.HIGHEST)
    a = jnp.maximum(_bdot(u, w1[...]) + b1[...], 0.0)
    a = jnp.maximum(_bdot(a, w2[...]) + b2[...], 0.0)
    a = jnp.maximum(_bdot(a, w3[...]) + b3[...], 0.0)
    out_ref[...] = jnp.maximum(_bdot(a, w4[...]) + b4[...], 0.0)


def _run_final(hl16, pool_pp, ws):
    ins = [hl16, pool_pp] + list(ws)
    return pl.pallas_call(
        _final_body,
        in_specs=[pl.BlockSpec(v.shape, lambda *_, n=v.ndim: (0,) * n) for v in ins],
        out_specs=pl.BlockSpec((NG, D), lambda *_: (0, 0)),
        out_shape=jax.ShapeDtypeStruct((NG, D), jnp.float32),
    )(*ins)


_NC, _NS = 2, 16
_NW = _NC * _NS
_EROWS = E // 128
_ROWS_W = _EROWS // _NW
_K = 8
_SLABS = _ROWS_W // _K
_ACCR = P + 512
_ZR = _ACCR // _NS


def _make_agg_sc(mode):
    mesh = plsc.VectorSubcoreMesh(core_axis_name="c", subcore_axis_name="s")

    @functools.partial(
        pl.kernel,
        out_type=jax.ShapeDtypeStruct((_NC, P, F), jnp.float32),
        mesh=mesh,
        compiler_params=pltpu.CompilerParams(use_tc_tiling_on_sc=False),
        scratch_types=[
            pltpu.VMEM((_K, 128), jnp.int32),
            pltpu.VMEM((_K, 128), jnp.int32),
            pltpu.VMEM((_K * 128, F), jnp.float32),
            pltpu.VMEM_SHARED((_ACCR, F), jnp.float32),
            pltpu.SemaphoreType.DMA,
        ])
    def agg(src_hbm, dst_hbm, h_hbm, out_hbm, srcv, dstv, rows, acc, sem):
        c = lax.axis_index("c")
        s = lax.axis_index("s")
        wid = s * _NC + c

        def zrow(i, _):
            rows[i, :] = jnp.zeros((F,), jnp.float32)
            return 0
        lax.fori_loop(0, 1024, zrow, 0)
        z0 = s * _ZR
        pltpu.sync_copy(rows.at[pl.ds(0, 1024)], acc.at[pl.ds(z0, 1024)])
        pltpu.sync_copy(rows.at[pl.ds(0, 1024)], acc.at[pl.ds(z0 + 1024, 1024)])
        pltpu.sync_copy(rows.at[pl.ds(0, 32)], acc.at[pl.ds(z0 + 2048, 32)])
        plsc.subcore_barrier()

        base_row = wid * _ROWS_W

        def slab(g, _):
            rb = base_row + g * _K
            pltpu.sync_copy(src_hbm.at[pl.ds(rb, _K)], srcv)
            pltpu.sync_copy(dst_hbm.at[pl.ds(rb, _K)], dstv)
            if mode != MODE_NONE:
                for j in range(_K):
                    for v in range(128 // F):
                        sl = pl.ds(v * F, F)
                        d = dstv[j, sl]
                        if mode == MODE_SUB:
                            d = d - P
                        dstv[j, sl] = jnp.where((d >= 0) & (d < P), d, P)
            cps = [pltpu.async_copy(h_hbm.at[srcv.at[j]],
                                    rows.at[pl.ds(j * 128, 128)], sem)
                   for j in range(_K)]
            for cp in cps:
                cp.wait()
            for j in range(_K):
                pltpu.sync_copy(rows.at[pl.ds(j * 128, 128)],
                                acc.at[dstv.at[j]], add=True)
            return 0
        lax.fori_loop(0, _SLABS, slab, 0)
        plsc.subcore_barrier()
        wb = s * (P // _NS)
        pltpu.sync_copy(acc.at[pl.ds(wb, P // _NS)],
                        out_hbm.at[c, pl.ds(wb, P // _NS)])
    return agg


_AGG_SC = {m: _make_agg_sc(m) for m in (MODE_NONE, MODE_SUB, MODE_MASK)}


def _agg(H, src_g, dst, mode):
    src2d = src_g.reshape(_EROWS, 128)
    dst2d = dst.reshape(_EROWS, 128)
    return _AGG_SC[mode](src2d, dst2d, H)


_NROWS_W = N // _NW
_BIROWS = N // 128


def _make_pool_sc():
    mesh = plsc.VectorSubcoreMesh(core_axis_name="c", subcore_axis_name="s")

    @functools.partial(
        pl.kernel,
        out_type=jax.ShapeDtypeStruct((_NC, NG, F), jnp.float32),
        mesh=mesh,
        compiler_params=pltpu.CompilerParams(use_tc_tiling_on_sc=False),
        scratch_types=[
            pltpu.VMEM((_K, 128), jnp.int32),
            pltpu.VMEM((_K * 128, F), jnp.float32),
            pltpu.VMEM_SHARED((NG, F), jnp.float32),
            pltpu.SemaphoreType.DMA,
        ])
    def pool(h_hbm, bidx_hbm, out_hbm, bidxv, rows, acc, sem):
        c = lax.axis_index("c")
        s = lax.axis_index("s")
        wid = s * _NC + c

        def zrow(i, _):
            rows[i, :] = jnp.zeros((F,), jnp.float32)
            return 0
        lax.fori_loop(0, NG // _NS, zrow, 0)
        pltpu.sync_copy(rows.at[pl.ds(0, NG // _NS)],
                        acc.at[pl.ds(s * (NG // _NS), NG // _NS)])
        plsc.subcore_barrier()

        def slab(g, _):
            node0 = wid * _NROWS_W + g * 1024
            pltpu.sync_copy(h_hbm.at[pl.ds(node0, 1024)], rows)
            pltpu.sync_copy(bidx_hbm.at[pl.ds(wid * (_NROWS_W // 128) + g * _K, _K)],
                            bidxv)
            for j in range(_K):
                pltpu.sync_copy(rows.at[pl.ds(j * 128, 128)],
                                acc.at[bidxv.at[j]], add=True)
            return 0
        lax.fori_loop(0, _NROWS_W // 1024, slab, 0)
        plsc.subcore_barrier()
        pltpu.sync_copy(acc.at[pl.ds(s * (NG // _NS), NG // _NS)],
                        out_hbm.at[c, pl.ds(s * (NG // _NS), NG // _NS)])
    return pool


_POOL_SC = _make_pool_sc()


def _pool(H, batch_idx):
    return _POOL_SC(H, batch_idx.reshape(_BIROWS, 128))


def _pad_mlp(layers):
    (w1, b1), (w2, b2), (w3, b3), (w4, b4) = layers
    w1p = jnp.zeros((F, D), jnp.float32).at[:w1.shape[0]].set(w1)
    w4p = jnp.zeros((D, F), jnp.float32).at[:, :DYN].set(w4)
    b4p = jnp.zeros((F,), jnp.float32).at[:DYN].set(b4)
    return (w1p, b1[None, :], w2, b2[None, :], w3, b3[None, :], w4p, b4p[None, :])


def kernel(x, feature_mtx_static, hlvs, params, batch_idx, inner_edges,
           forward_edges, backward_edges):
    ws_in = _pad_mlp(params['inlayer_mlp'])
    ws_fw = _pad_mlp(params['forward_mlp'])
    ws_bw = _pad_mlp(params['backward_mlp'])
    ws_nd = _pad_mlp(params['node_dnn'])
    eps_in = (params['inlayer_eps'])[None].astype(jnp.float32)
    eps_fw = (params['forward_eps'])[None].astype(jnp.float32)
    eps_bw = (params['backward_eps'])[None].astype(jnp.float32)

    tbl = jnp.zeros((NG, F), jnp.float32).at[:, DYN + STAT:DYN + STAT + NHL].set(hlvs)
    hlp = tbl[batch_idx]
    H = _run_init_h(x, feature_mtx_static, hlp)

    def conv(H, edges, lo, mode, ws, eps, base):
        src_g = edges[0] + lo
        pp = _agg(H, src_g, edges[1], mode)
        return _run_mlp_gin(H, pp, eps, ws, base)

    H = conv(H, inner_edges[0], 0, MODE_NONE, ws_in, eps_in, 0)
    H = conv(H, forward_edges[0], 0, MODE_SUB, ws_fw, eps_fw, P)
    H = _run_mlp_plain(H, ws_nd, P)
    H = conv(H, inner_edges[1], P, MODE_NONE, ws_in, eps_in, P)
    H = conv(H, forward_edges[1], P, MODE_SUB, ws_fw, eps_fw, 2 * P)
    H = _run_mlp_plain(H, ws_nd, 2 * P)
    H = conv(H, inner_edges[2], 2 * P, MODE_NONE, ws_in, eps_in, 2 * P)
    H = conv(H, backward_edges[1], P, MODE_MASK, ws_bw, eps_bw, P)
    H = conv(H, inner_edges[1], P, MODE_NONE, ws_in, eps_in, P)
    H = _run_mlp_plain(H, ws_nd, P)
    H = conv(H, backward_edges[0], 0, MODE_MASK, ws_bw, eps_bw, 0)
    H = conv(H, inner_edges[0], 0, MODE_NONE, ws_in, eps_in, 0)
    H = _run_mlp_plain(H, ws_nd, 0)

    pool_pp = _pool(H, batch_idx)

    (w1, b1), (w2, b2), (w3, b3), (w4, b4) = params['hlv_dnn']
    w1p = jnp.zeros((F, D), jnp.float32).at[:NHL + DYN].set(w1)
    shift = jnp.zeros((F, F), jnp.float32)
    for j in range(DYN):
        shift = shift.at[j, NHL + j].set(1.0)
    w4p = jnp.zeros((D, D), jnp.float32).at[:, :1].set(w4)
    hl16 = jnp.zeros((NG, F), jnp.float32).at[:, :NHL].set(hlvs)
    fin = _run_final(hl16, pool_pp,
                     (shift, w1p, b1[None, :], w2, b2[None, :], w3, b3[None, :],
                      w4p, jnp.zeros((1, D), jnp.float32).at[0, 0].set(b4[0])))
    return fin[:, :1]


# speedup vs baseline: 17.0511x; 1.1177x over previous
import functools

import jax
import jax.numpy as jnp
from jax import lax
from jax.experimental import pallas as pl
from jax.experimental.pallas import tpu as pltpu
from jax.experimental.pallas import tpu_sc as plsc

P = 32768
NL = 3
N = P * NL
NG = 256
DYN = 4
STAT = 5
NHL = 6
D = 128
E = 1048576
F = 16
BLK = 2048

def _bdot(a, b):
    return jnp.dot(a.astype(jnp.bfloat16), b.astype(jnp.bfloat16),
                   preferred_element_type=jnp.float32)


def _stat_mask():
    i = lax.broadcasted_iota(jnp.int32, (1, F), 1)
    return jnp.where((i >= DYN) & (i < F - 1), 1.0, 0.0).astype(jnp.float32)

MODE_NONE = 0
MODE_SUB = 1
MODE_MASK = 2


def _mlp_gin_body(eps_ref, h_ref, pp_ref, w1, b1, w2, b2, w3, b3, w4, b4,
                  out_ref):
    u = (1.0 + eps_ref[0]) * h_ref[...] + pp_ref[0] + pp_ref[1]
    a = jnp.maximum(_bdot(u, w1[...]) + b1[...], 0.0)
    a = jnp.maximum(_bdot(a, w2[...]) + b2[...], 0.0)
    a = jnp.maximum(_bdot(a, w3[...]) + b3[...], 0.0)
    o = jnp.maximum(_bdot(a, w4[...]) + b4[...], 0.0)
    out_ref[...] = o + h_ref[...] * _stat_mask()


def _mlp_plain_body(h_ref, w1, b1, w2, b2, w3, b3, w4, b4, out_ref):
    a = jnp.maximum(_bdot(h_ref[...], w1[...]) + b1[...], 0.0)
    a = jnp.maximum(_bdot(a, w2[...]) + b2[...], 0.0)
    a = jnp.maximum(_bdot(a, w3[...]) + b3[...], 0.0)
    o = jnp.maximum(_bdot(a, w4[...]) + b4[...], 0.0)
    out_ref[...] = o + h_ref[...] * _stat_mask()


def _w_specs(ws):
    return [pl.BlockSpec(w.shape, lambda i, n=w.ndim: (0,) * n) for w in ws]


def _run_mlp_gin(H, pp, eps, ws, base):
    b0 = base // BLK
    row_spec = pl.BlockSpec((BLK, F), lambda i, b0=b0: (b0 + i, 0))
    return pl.pallas_call(
        _mlp_gin_body,
        grid=(P // BLK,),
        in_specs=[pl.BlockSpec(memory_space=pltpu.SMEM),
                  row_spec,
                  pl.BlockSpec((2, BLK, F), lambda i: (0, i, 0))]
                 + _w_specs(ws),
        out_specs=row_spec,
        out_shape=jax.ShapeDtypeStruct((N, F), jnp.float32),
        input_output_aliases={1: 0},
    )(eps, H, pp, *ws)


def _run_mlp_plain(H, ws, base):
    b0 = base // BLK
    row_spec = pl.BlockSpec((BLK, F), lambda i, b0=b0: (b0 + i, 0))
    return pl.pallas_call(
        _mlp_plain_body,
        grid=(P // BLK,),
        in_specs=[row_spec] + _w_specs(ws),
        out_specs=row_spec,
        out_shape=jax.ShapeDtypeStruct((N, F), jnp.float32),
        input_output_aliases={0: 0},
    )(H, *ws)


def _init_h_body(x_ref, fs_ref, hlp_ref, out_ref):
    col = lax.broadcasted_iota(jnp.int32, (1, F), 1)
    e0 = jnp.where(col == 0, 1.0, 0.0).astype(jnp.float32)
    r = lax.broadcasted_iota(jnp.int32, (STAT, F), 0)
    c = lax.broadcasted_iota(jnp.int32, (STAT, F), 1)
    s = jnp.where(c == r + DYN, 1.0, 0.0).astype(jnp.float32)
    out_ref[...] = (x_ref[...] * e0
                    + jnp.dot(fs_ref[...], s, preferred_element_type=jnp.float32, precision=lax.Precision.HIGHEST)
                    + hlp_ref[...])


def _run_init_h(x, fstat, hlp):
    return pl.pallas_call(
        _init_h_body,
        grid=(N // BLK,),
        in_specs=[pl.BlockSpec((BLK, 1), lambda i: (i, 0)),
                  pl.BlockSpec((BLK, STAT), lambda i: (i, 0)),
                  pl.BlockSpec((BLK, F), lambda i: (i, 0))],
        out_specs=pl.BlockSpec((BLK, F), lambda i: (i, 0)),
        out_shape=jax.ShapeDtypeStruct((N, F), jnp.float32),
    )(x, fstat, hlp)


def _final_body(hl_ref, pp_ref, shift, w1, b1, w2, b2, w3, b3, w4, b4, out_ref):
    pooled = pp_ref[0] + pp_ref[1]
    u = hl_ref[...] + jnp.dot(pooled, shift[...],
                              preferred_element_type=jnp.float32,
                              precision=lax.Precision.HIGHEST)
    a = jnp.maximum(_bdot(u, w1[...]) + b1[...], 0.0)
    a = jnp.maximum(_bdot(a, w2[...]) + b2[...], 0.0)
    a = jnp.maximum(_bdot(a, w3[...]) + b3[...], 0.0)
    out_ref[...] = jnp.maximum(_bdot(a, w4[...]) + b4[...], 0.0)


def _run_final(hl16, pool_pp, ws):
    ins = [hl16, pool_pp] + list(ws)
    return pl.pallas_call(
        _final_body,
        in_specs=[pl.BlockSpec(v.shape, lambda *_, n=v.ndim: (0,) * n) for v in ins],
        out_specs=pl.BlockSpec((NG, D), lambda *_: (0, 0)),
        out_shape=jax.ShapeDtypeStruct((NG, D), jnp.float32),
    )(*ins)


_NC, _NS = 2, 16
_NW = _NC * _NS
_EROWS = E // 128
_ROWS_W = _EROWS // _NW
_K = 16
_SLABS = _ROWS_W // _K
_PAIRS = _SLABS // 2
_ACCR = P + 512
_ZR = _ACCR // _NS
_SROWS = _K * 128


def _make_agg_sc(mode):
    mesh = plsc.VectorSubcoreMesh(core_axis_name="c", subcore_axis_name="s")

    @functools.partial(
        pl.kernel,
        out_type=jax.ShapeDtypeStruct((_NC, P, F), jnp.float32),
        mesh=mesh,
        compiler_params=pltpu.CompilerParams(use_tc_tiling_on_sc=False),
        scratch_types=[
            pltpu.VMEM((_K, 128), jnp.int32),
            pltpu.VMEM((_K, 128), jnp.int32),
            pltpu.VMEM((_K, 128), jnp.int32),
            pltpu.VMEM((_K, 128), jnp.int32),
            pltpu.VMEM((_SROWS, F), jnp.float32),
            pltpu.VMEM((_SROWS, F), jnp.float32),
            pltpu.VMEM_SHARED((_ACCR, F), jnp.float32),
            pltpu.SemaphoreType.DMA,
            pltpu.SemaphoreType.DMA,
            pltpu.SemaphoreType.DMA,
            pltpu.SemaphoreType.DMA,
            pltpu.SemaphoreType.DMA,
            pltpu.SemaphoreType.DMA,
        ])
    def agg(src_hbm, dst_hbm, h_hbm, out_hbm,
            srcv0, srcv1, dstv0, dstv1, rows0, rows1, acc,
            semi0, semi1, semg0, semg1, semw0, semw1):
        c = lax.axis_index("c")
        s = lax.axis_index("s")
        wid = s * _NC + c
        srcv = (srcv0, srcv1)
        dstv = (dstv0, dstv1)
        rows = (rows0, rows1)
        semi = (semi0, semi1)
        semg = (semg0, semg1)
        semw = (semw0, semw1)
        base_row = wid * _ROWS_W

        def transform(dv):
            if mode != MODE_NONE:
                for j in range(_K):
                    for v in range(128 // F):
                        sl = pl.ds(v * F, F)
                        d = dv[j, sl]
                        if mode == MODE_SUB:
                            d = d - P
                        dv[j, sl] = jnp.where((d >= 0) & (d < P), d, P)

        def issue_idx(rb, b):
            return (pltpu.async_copy(src_hbm.at[pl.ds(rb, _K)], srcv[b], semi[b]),
                    pltpu.async_copy(dst_hbm.at[pl.ds(rb, _K)], dstv[b], semi[b]))

        def fire_gathers(b):
            for j in range(_K):
                pltpu.async_copy(h_hbm.at[srcv[b].at[j]],
                                 rows[b].at[pl.ds(j * 128, 128)], semg[b])

        def fire_scatters(b):
            for j in range(_K):
                pltpu.async_copy(rows[b].at[pl.ds(j * 128, 128)],
                                 acc.at[dstv[b].at[j]], semw[b], add=True)

        def drain_gathers(b):
            pltpu.make_async_copy(h_hbm.at[pl.ds(0, _SROWS)], rows[b],
                                  semg[b]).wait()

        def drain_scatters(b):
            pltpu.make_async_copy(rows[b], acc.at[pl.ds(0, _SROWS)],
                                  semw[b]).wait()

        i0 = issue_idx(base_row, 0)
        i1 = issue_idx(base_row + _K, 1)

        def zrow(i, _):
            rows0[i, :] = jnp.zeros((F,), jnp.float32)
            return 0
        lax.fori_loop(0, _SROWS, zrow, 0)
        z0 = s * _ZR
        pltpu.sync_copy(rows0.at[pl.ds(0, 2048)], acc.at[pl.ds(z0, 2048)])
        pltpu.sync_copy(rows0.at[pl.ds(0, 32)], acc.at[pl.ds(z0 + 2048, 32)])
        plsc.subcore_barrier()

        for cp in i0:
            cp.wait()
        transform(dstv0)
        fire_gathers(0)

        def do_slab(g, b):
            drain_scatters(b)
            ii = issue_idx(g * _K + base_row, b)
            drain_gathers(1 - b)
            fire_scatters(1 - b)
            for cp in ii:
                cp.wait()
            transform(dstv[b])
            fire_gathers(b)

        def pair(i, _):
            do_slab(2 * i, 0)
            do_slab(2 * i + 1, 1)
            return 0

        drain_gathers(0)
        for cp in i1:
            cp.wait()
        transform(dstv1)
        fire_scatters(0)
        fire_gathers(1)
        lax.fori_loop(1, _PAIRS, pair, 0)
        drain_scatters(0)
        drain_gathers(1)
        fire_scatters(1)
        drain_scatters(1)
        plsc.subcore_barrier()
        wb = s * (P // _NS)
        pltpu.sync_copy(acc.at[pl.ds(wb, P // _NS)],
                        out_hbm.at[c, pl.ds(wb, P // _NS)])
    return agg


_AGG_SC = {m: _make_agg_sc(m) for m in (MODE_NONE, MODE_SUB, MODE_MASK)}


def _agg(H, src_g, dst, mode):
    src2d = src_g.reshape(_EROWS, 128)
    dst2d = dst.reshape(_EROWS, 128)
    return _AGG_SC[mode](src2d, dst2d, H)


_NROWS_W = N // _NW
_BIROWS = N // 128


def _make_pool_sc():
    mesh = plsc.VectorSubcoreMesh(core_axis_name="c", subcore_axis_name="s")

    @functools.partial(
        pl.kernel,
        out_type=jax.ShapeDtypeStruct((_NC, NG, F), jnp.float32),
        mesh=mesh,
        compiler_params=pltpu.CompilerParams(use_tc_tiling_on_sc=False),
        scratch_types=[
            pltpu.VMEM((8, 128), jnp.int32),
            pltpu.VMEM((1024, F), jnp.float32),
            pltpu.VMEM_SHARED((NG, F), jnp.float32),
            pltpu.SemaphoreType.DMA,
        ])
    def pool(h_hbm, bidx_hbm, out_hbm, bidxv, rows, acc, sem):
        c = lax.axis_index("c")
        s = lax.axis_index("s")
        wid = s * _NC + c

        def zrow(i, _):
            rows[i, :] = jnp.zeros((F,), jnp.float32)
            return 0
        lax.fori_loop(0, NG // _NS, zrow, 0)
        pltpu.sync_copy(rows.at[pl.ds(0, NG // _NS)],
                        acc.at[pl.ds(s * (NG // _NS), NG // _NS)])
        plsc.subcore_barrier()

        def slab(g, _):
            node0 = wid * _NROWS_W + g * 1024
            pltpu.sync_copy(h_hbm.at[pl.ds(node0, 1024)], rows)
            pltpu.sync_copy(bidx_hbm.at[pl.ds(wid * (_NROWS_W // 128) + g * 8, 8)],
                            bidxv)
            for j in range(8):
                pltpu.sync_copy(rows.at[pl.ds(j * 128, 128)],
                                acc.at[bidxv.at[j]], add=True)
            return 0
        lax.fori_loop(0, _NROWS_W // 1024, slab, 0)
        plsc.subcore_barrier()
        pltpu.sync_copy(acc.at[pl.ds(s * (NG // _NS), NG // _NS)],
                        out_hbm.at[c, pl.ds(s * (NG // _NS), NG // _NS)])
    return pool


_POOL_SC = _make_pool_sc()


def _pool(H, batch_idx):
    return _POOL_SC(H, batch_idx.reshape(_BIROWS, 128))


def _pad_mlp(layers):
    (w1, b1), (w2, b2), (w3, b3), (w4, b4) = layers
    w1p = jnp.zeros((F, D), jnp.float32).at[:w1.shape[0]].set(w1)
    w4p = jnp.zeros((D, F), jnp.float32).at[:, :DYN].set(w4)
    b4p = jnp.zeros((F,), jnp.float32).at[:DYN].set(b4)
    return (w1p, b1[None, :], w2, b2[None, :], w3, b3[None, :], w4p, b4p[None, :])


def kernel(x, feature_mtx_static, hlvs, params, batch_idx, inner_edges,
           forward_edges, backward_edges):
    ws_in = _pad_mlp(params['inlayer_mlp'])
    ws_fw = _pad_mlp(params['forward_mlp'])
    ws_bw = _pad_mlp(params['backward_mlp'])
    ws_nd = _pad_mlp(params['node_dnn'])
    eps_in = (params['inlayer_eps'])[None].astype(jnp.float32)
    eps_fw = (params['forward_eps'])[None].astype(jnp.float32)
    eps_bw = (params['backward_eps'])[None].astype(jnp.float32)

    tbl = jnp.zeros((NG, F), jnp.float32).at[:, DYN + STAT:DYN + STAT + NHL].set(hlvs)
    hlp = tbl[batch_idx]
    H = _run_init_h(x, feature_mtx_static, hlp)

    def conv(H, edges, lo, mode, ws, eps, base):
        src_g = edges[0] + lo
        pp = _agg(H, src_g, edges[1], mode)
        return _run_mlp_gin(H, pp, eps, ws, base)

    H = conv(H, inner_edges[0], 0, MODE_NONE, ws_in, eps_in, 0)
    H = conv(H, forward_edges[0], 0, MODE_SUB, ws_fw, eps_fw, P)
    H = _run_mlp_plain(H, ws_nd, P)
    H = conv(H, inner_edges[1], P, MODE_NONE, ws_in, eps_in, P)
    H = conv(H, forward_edges[1], P, MODE_SUB, ws_fw, eps_fw, 2 * P)
    H = _run_mlp_plain(H, ws_nd, 2 * P)
    H = conv(H, inner_edges[2], 2 * P, MODE_NONE, ws_in, eps_in, 2 * P)
    H = conv(H, backward_edges[1], P, MODE_MASK, ws_bw, eps_bw, P)
    H = conv(H, inner_edges[1], P, MODE_NONE, ws_in, eps_in, P)
    H = _run_mlp_plain(H, ws_nd, P)
    H = conv(H, backward_edges[0], 0, MODE_MASK, ws_bw, eps_bw, 0)
    H = conv(H, inner_edges[0], 0, MODE_NONE, ws_in, eps_in, 0)
    H = _run_mlp_plain(H, ws_nd, 0)

    pool_pp = _pool(H, batch_idx)

    (w1, b1), (w2, b2), (w3, b3), (w4, b4) = params['hlv_dnn']
    w1p = jnp.zeros((F, D), jnp.float32).at[:NHL + DYN].set(w1)
    shift = jnp.zeros((F, F), jnp.float32)
    for j in range(DYN):
        shift = shift.at[j, NHL + j].set(1.0)
    w4p = jnp.zeros((D, D), jnp.float32).at[:, :1].set(w4)
    hl16 = jnp.zeros((NG, F), jnp.float32).at[:, :NHL].set(hlvs)
    fin = _run_final(hl16, pool_pp,
                     (shift, w1p, b1[None, :], w2, b2[None, :], w3, b3[None, :],
                      w4p, jnp.zeros((1, D), jnp.float32).at[0, 0].set(b4[0])))
    return fin[:, :1]
